# Initial kernel scaffold; baseline (speedup 1.0000x reference)
import functools

import jax
import jax.numpy as jnp
from jax import lax
from jax.experimental import pallas as pl
from jax.experimental.pallas import tpu as pltpu
from jax.experimental.pallas import tpu_sc as plsc

NC = 2
NS = 16
NW = NC * NS
CHUNK = 128


def _tile_rows(n):
    rpt = (n // NS) // 8 * 8
    rem = n - rpt * NS
    return rpt, rem


def _hist_body(nf, rchunks, src2, dst2, zcol, ones_col, out,
               sidx, didx, ones_v, hist_s, hist_d):
    n = hist_s.shape[0]
    c = lax.axis_index("c")
    s = lax.axis_index("s")
    wid = s * NC + c
    rpt, rem = _tile_rows(n)

    pltpu.sync_copy(zcol.at[pl.ds(0, rpt)], hist_s.at[pl.ds(s * rpt, rpt)])
    pltpu.sync_copy(zcol.at[pl.ds(0, rpt)], hist_d.at[pl.ds(s * rpt, rpt)])

    @pl.when(s == NS - 1)
    def _():
        pltpu.sync_copy(zcol.at[pl.ds(0, rem)], hist_s.at[pl.ds(NS * rpt, rem)])
        pltpu.sync_copy(zcol.at[pl.ds(0, rem)], hist_d.at[pl.ds(NS * rpt, rem)])

    pltpu.sync_copy(ones_col, ones_v)
    plsc.subcore_barrier()

    def step(chunk):
        pltpu.sync_copy(src2.at[chunk], sidx)
        pltpu.sync_copy(dst2.at[chunk], didx)
        pltpu.sync_copy(ones_v, hist_s.at[sidx], add=True)
        pltpu.sync_copy(ones_v, hist_d.at[didx], add=True)

    def body(i, carry):
        step(wid + i * NW)
        return carry

    lax.fori_loop(0, nf, body, 0)

    @pl.when(wid < rchunks)
    def _():
        step(nf * NW + wid)

    plsc.subcore_barrier()

    pltpu.sync_copy(hist_s.at[pl.ds(s * rpt, rpt)],
                    out.at[c, 0, pl.ds(s * rpt, rpt)])
    pltpu.sync_copy(hist_d.at[pl.ds(s * rpt, rpt)],
                    out.at[c, 1, pl.ds(s * rpt, rpt)])

    @pl.when(s == NS - 1)
    def _():
        pltpu.sync_copy(hist_s.at[pl.ds(NS * rpt, rem)],
                        out.at[c, 0, pl.ds(NS * rpt, rem)])
        pltpu.sync_copy(hist_d.at[pl.ds(NS * rpt, rem)],
                        out.at[c, 1, pl.ds(NS * rpt, rem)])


def _sc_hist(src2, dst2, n):
    nchunks = src2.shape[0]
    nf, rchunks = nchunks // NW, nchunks % NW
    rpt, _ = _tile_rows(n)
    zcol = jnp.zeros((rpt, 1), jnp.float32)
    ones_col = jnp.ones((CHUNK, 1), jnp.float32)
    mesh = plsc.VectorSubcoreMesh(core_axis_name="c", subcore_axis_name="s")
    return pl.kernel(
        functools.partial(_hist_body, nf, rchunks),
        out_type=jax.ShapeDtypeStruct((NC, 2, n, 1), jnp.float32),
        mesh=mesh,
        scratch_types=[
            pltpu.VMEM((CHUNK,), jnp.int32),
            pltpu.VMEM((CHUNK,), jnp.int32),
            pltpu.VMEM((CHUNK, 1), jnp.float32),
            pltpu.VMEM_SHARED((n, 1), jnp.float32),
            pltpu.VMEM_SHARED((n, 1), jnp.float32),
        ],
    )(src2, dst2, zcol, ones_col)


def _agg_body(nf, rchunks, src2, dst2, y2, zrows, out,
              sidx, didx, rows, agg):
    n, d = agg.shape
    c = lax.axis_index("c")
    s = lax.axis_index("s")
    wid = s * NC + c
    rpt, rem = _tile_rows(n)

    pltpu.sync_copy(zrows.at[pl.ds(0, rpt)], agg.at[pl.ds(s * rpt, rpt)])

    @pl.when(s == NS - 1)
    def _():
        pltpu.sync_copy(zrows.at[pl.ds(0, rem)], agg.at[pl.ds(NS * rpt, rem)])

    plsc.subcore_barrier()

    def step(chunk):
        pltpu.sync_copy(src2.at[chunk], sidx)
        pltpu.sync_copy(dst2.at[chunk], didx)
        pltpu.sync_copy(y2.at[sidx], rows)
        pltpu.sync_copy(rows, agg.at[didx], add=True)

    def body(i, carry):
        step(wid + i * NW)
        return carry

    lax.fori_loop(0, nf, body, 0)

    @pl.when(wid < rchunks)
    def _():
        step(nf * NW + wid)

    plsc.subcore_barrier()

    pltpu.sync_copy(agg.at[pl.ds(s * rpt, rpt)],
                    out.at[c, pl.ds(s * rpt, rpt)])

    @pl.when(s == NS - 1)
    def _():
        pltpu.sync_copy(agg.at[pl.ds(NS * rpt, rem)],
                        out.at[c, pl.ds(NS * rpt, rem)])


def _sc_aggregate(src2, dst2, y2):
    nchunks = src2.shape[0]
    n, d = y2.shape
    nf, rchunks = nchunks // NW, nchunks % NW
    rpt, _ = _tile_rows(n)
    zrows = jnp.zeros((rpt, d), jnp.float32)
    mesh = plsc.VectorSubcoreMesh(core_axis_name="c", subcore_axis_name="s")
    return pl.kernel(
        functools.partial(_agg_body, nf, rchunks),
        out_type=jax.ShapeDtypeStruct((NC, n, d), jnp.float32),
        mesh=mesh,
        scratch_types=[
            pltpu.VMEM((CHUNK,), jnp.int32),
            pltpu.VMEM((CHUNK,), jnp.int32),
            pltpu.VMEM((CHUNK, d), jnp.float32),
            pltpu.VMEM_SHARED((n, d), jnp.float32),
        ],
    )(src2, dst2, y2, zrows)


def _dinv(deg):
    return jnp.where(deg > 0, lax.rsqrt(jnp.maximum(deg, 1e-12)), 0.0)


def _matmul_scale_body(x_ref, w_ref, hist_ref, y2_ref):
    deg_src = hist_ref[0, 0] + hist_ref[1, 0]
    y = jnp.dot(x_ref[...], w_ref[...], preferred_element_type=jnp.float32)
    y2_ref[...] = y * _dinv(deg_src)


def _tc_matmul_scale(x, w, hist, blk=2000):
    n, dout = x.shape[0], w.shape[1]
    grid = n // blk
    return pl.pallas_call(
        _matmul_scale_body,
        grid=(grid,),
        in_specs=[
            pl.BlockSpec((blk, x.shape[1]), lambda i: (i, 0)),
            pl.BlockSpec((w.shape[0], dout), lambda i: (0, 0)),
            pl.BlockSpec((NC, 2, blk, 1), lambda i: (0, 0, i, 0)),
        ],
        out_specs=pl.BlockSpec((blk, dout), lambda i: (i, 0)),
        out_shape=jax.ShapeDtypeStruct((n, dout), jnp.float32),
    )(x, w, hist)


def _final_body(parts_ref, hist_ref, b_ref, out_ref):
    deg_dst = hist_ref[0, 1] + hist_ref[1, 1]
    agg = parts_ref[0] + parts_ref[1]
    out_ref[...] = agg * _dinv(deg_dst) + b_ref[...]


def _tc_final(parts, hist, b, blk=2000):
    n, d = parts.shape[1], parts.shape[2]
    grid = n // blk
    return pl.pallas_call(
        _final_body,
        grid=(grid,),
        in_specs=[
            pl.BlockSpec((NC, blk, d), lambda i: (0, i, 0)),
            pl.BlockSpec((NC, 2, blk, 1), lambda i: (0, 0, i, 0)),
            pl.BlockSpec((1, d), lambda i: (0, 0)),
        ],
        out_specs=pl.BlockSpec((blk, d), lambda i: (i, 0)),
        out_shape=jax.ShapeDtypeStruct((n, d), jnp.float32),
    )(parts, hist, b.reshape(1, d))


@jax.jit
def kernel(x, edge_index, W, b):
    n = x.shape[0]
    e = edge_index.shape[1]
    src2 = edge_index[0].reshape(e // CHUNK, CHUNK)
    dst2 = edge_index[1].reshape(e // CHUNK, CHUNK)

    hist = _sc_hist(src2, dst2, n)
    y2 = _tc_matmul_scale(x, W, hist)
    parts = _sc_aggregate(src2, dst2, y2)
    return _tc_final(parts, hist, b)


# speedup vs baseline: 17.3135x; 17.3135x over previous
import functools

import jax
import jax.numpy as jnp
from jax import lax
from jax.experimental import pallas as pl
from jax.experimental.pallas import tpu as pltpu
from jax.experimental.pallas import tpu_sc as plsc

NC = 2
NS = 16
NW = NC * NS
CHUNK = 128


def _tile_rows(n):
    rpt = (n // NS) // 8 * 8
    rem = n - rpt * NS
    return rpt, rem


def _hist_body(nf, rchunks, src2, dst2, zeros_h, row_ids, out,
               sidx, didx, hs, hd, ridx, sh_s, sh_d):
    nrows = sh_s.shape[0]
    c = lax.axis_index("c")
    s = lax.axis_index("s")
    wid = s * NC + c
    nw8 = nrows // 8

    pltpu.sync_copy(zeros_h, hs)
    pltpu.sync_copy(zeros_h, hd)

    @pl.when(s < nw8)
    def _():
        pltpu.sync_copy(zeros_h.at[pl.ds(0, 8)], sh_s.at[pl.ds(s * 8, 8)])
        pltpu.sync_copy(zeros_h.at[pl.ds(0, 8)], sh_d.at[pl.ds(s * 8, 8)])

    pltpu.sync_copy(row_ids, ridx)
    plsc.subcore_barrier()

    ones16 = jnp.ones((16,), jnp.float32)

    def scat(hist, iv):
        plsc.addupdate_scatter(
            hist, [lax.shift_right_logical(iv, 7), lax.bitwise_and(iv, 127)],
            ones16)

    def step(chunk):
        pltpu.sync_copy(src2.at[chunk], sidx)
        pltpu.sync_copy(dst2.at[chunk], didx)
        def inner(j, carry):
            scat(hs, sidx[pl.ds(j * 16, 16)])
            scat(hd, didx[pl.ds(j * 16, 16)])
            return carry
        lax.fori_loop(0, CHUNK // 16, inner, 0)

    def body(i, carry):
        step(wid + i * NW)
        return carry

    lax.fori_loop(0, nf, body, 0)

    @pl.when(wid < rchunks)
    def _():
        step(nf * NW + wid)

    pltpu.sync_copy(hs, sh_s.at[ridx], add=True)
    pltpu.sync_copy(hd, sh_d.at[ridx], add=True)
    plsc.subcore_barrier()

    @pl.when(s < nw8)
    def _():
        pltpu.sync_copy(sh_s.at[pl.ds(s * 8, 8)], out.at[c, 0, pl.ds(s * 8, 8)])
        pltpu.sync_copy(sh_d.at[pl.ds(s * 8, 8)], out.at[c, 1, pl.ds(s * 8, 8)])


def _sc_hist(src2, dst2, n):
    nchunks = src2.shape[0]
    nf, rchunks = nchunks // NW, nchunks % NW
    nrows = -(-n // (128 * NS)) * NS
    zeros_h = jnp.zeros((nrows, 128), jnp.float32)
    row_ids = jnp.arange(nrows, dtype=jnp.int32)
    mesh = plsc.VectorSubcoreMesh(core_axis_name="c", subcore_axis_name="s")
    hist = pl.kernel(
        functools.partial(_hist_body, nf, rchunks),
        out_type=jax.ShapeDtypeStruct((NC, 2, nrows, 128), jnp.float32),
        mesh=mesh,
        scratch_types=[
            pltpu.VMEM((CHUNK,), jnp.int32),
            pltpu.VMEM((CHUNK,), jnp.int32),
            pltpu.VMEM((nrows, 128), jnp.float32),
            pltpu.VMEM((nrows, 128), jnp.float32),
            pltpu.VMEM((nrows,), jnp.int32),
            pltpu.VMEM_SHARED((nrows, 128), jnp.float32),
            pltpu.VMEM_SHARED((nrows, 128), jnp.float32),
        ],
        compiler_params=pltpu.CompilerParams(needs_layout_passes=False),
    )(src2, dst2, zeros_h, row_ids)
    return hist.reshape(NC, 2, nrows * 128)[:, :, :n, None]


def _agg_body(nf, rchunks, src2, dst2, y2, zrows, out,
              sidx, didx, rows, agg):
    n, d = agg.shape
    c = lax.axis_index("c")
    s = lax.axis_index("s")
    wid = s * NC + c
    rpt, rem = _tile_rows(n)

    pltpu.sync_copy(zrows.at[pl.ds(0, rpt)], agg.at[pl.ds(s * rpt, rpt)])

    @pl.when(s == NS - 1)
    def _():
        pltpu.sync_copy(zrows.at[pl.ds(0, rem)], agg.at[pl.ds(NS * rpt, rem)])

    plsc.subcore_barrier()

    def step(chunk):
        pltpu.sync_copy(src2.at[chunk], sidx)
        pltpu.sync_copy(dst2.at[chunk], didx)
        pltpu.sync_copy(y2.at[sidx], rows)
        pltpu.sync_copy(rows, agg.at[didx], add=True)

    def body(i, carry):
        step(wid + i * NW)
        return carry

    lax.fori_loop(0, nf, body, 0)

    @pl.when(wid < rchunks)
    def _():
        step(nf * NW + wid)

    plsc.subcore_barrier()

    pltpu.sync_copy(agg.at[pl.ds(s * rpt, rpt)],
                    out.at[c, pl.ds(s * rpt, rpt)])

    @pl.when(s == NS - 1)
    def _():
        pltpu.sync_copy(agg.at[pl.ds(NS * rpt, rem)],
                        out.at[c, pl.ds(NS * rpt, rem)])


def _sc_aggregate(src2, dst2, y2):
    nchunks = src2.shape[0]
    n, d = y2.shape
    nf, rchunks = nchunks // NW, nchunks % NW
    rpt, _ = _tile_rows(n)
    zrows = jnp.zeros((rpt, d), jnp.float32)
    mesh = plsc.VectorSubcoreMesh(core_axis_name="c", subcore_axis_name="s")
    return pl.kernel(
        functools.partial(_agg_body, nf, rchunks),
        out_type=jax.ShapeDtypeStruct((NC, n, d), jnp.float32),
        mesh=mesh,
        scratch_types=[
            pltpu.VMEM((CHUNK,), jnp.int32),
            pltpu.VMEM((CHUNK,), jnp.int32),
            pltpu.VMEM((CHUNK, d), jnp.float32),
            pltpu.VMEM_SHARED((n, d), jnp.float32),
        ],
    )(src2, dst2, y2, zrows)


def _dinv(deg):
    return jnp.where(deg > 0, lax.rsqrt(jnp.maximum(deg, 1e-12)), 0.0)


def _matmul_scale_body(x_ref, w_ref, hist_ref, y2_ref):
    deg_src = hist_ref[0, 0] + hist_ref[1, 0]
    y = jnp.dot(x_ref[...], w_ref[...], preferred_element_type=jnp.float32)
    y2_ref[...] = y * _dinv(deg_src)


def _tc_matmul_scale(x, w, hist, blk=2000):
    n, dout = x.shape[0], w.shape[1]
    grid = n // blk
    return pl.pallas_call(
        _matmul_scale_body,
        grid=(grid,),
        in_specs=[
            pl.BlockSpec((blk, x.shape[1]), lambda i: (i, 0)),
            pl.BlockSpec((w.shape[0], dout), lambda i: (0, 0)),
            pl.BlockSpec((NC, 2, blk, 1), lambda i: (0, 0, i, 0)),
        ],
        out_specs=pl.BlockSpec((blk, dout), lambda i: (i, 0)),
        out_shape=jax.ShapeDtypeStruct((n, dout), jnp.float32),
    )(x, w, hist)


def _final_body(parts_ref, hist_ref, b_ref, out_ref):
    deg_dst = hist_ref[0, 1] + hist_ref[1, 1]
    agg = parts_ref[0] + parts_ref[1]
    out_ref[...] = agg * _dinv(deg_dst) + b_ref[...]


def _tc_final(parts, hist, b, blk=2000):
    n, d = parts.shape[1], parts.shape[2]
    grid = n // blk
    return pl.pallas_call(
        _final_body,
        grid=(grid,),
        in_specs=[
            pl.BlockSpec((NC, blk, d), lambda i: (0, i, 0)),
            pl.BlockSpec((NC, 2, blk, 1), lambda i: (0, 0, i, 0)),
            pl.BlockSpec((1, d), lambda i: (0, 0)),
        ],
        out_specs=pl.BlockSpec((blk, d), lambda i: (i, 0)),
        out_shape=jax.ShapeDtypeStruct((n, d), jnp.float32),
    )(parts, hist, b.reshape(1, d))


@jax.jit
def kernel(x, edge_index, W, b):
    n = x.shape[0]
    e = edge_index.shape[1]
    src2 = edge_index[0].reshape(e // CHUNK, CHUNK)
    dst2 = edge_index[1].reshape(e // CHUNK, CHUNK)

    hist = _sc_hist(src2, dst2, n)
    y2 = _tc_matmul_scale(x, W, hist)
    parts = _sc_aggregate(src2, dst2, y2)
    return _tc_final(parts, hist, b)


# speedup vs baseline: 28.1585x; 1.6264x over previous
import functools

import jax
import jax.numpy as jnp
from jax import lax
from jax.experimental import pallas as pl
from jax.experimental.pallas import tpu as pltpu
from jax.experimental.pallas import tpu_sc as plsc

NC = 2
NS = 16
NW = NC * NS
CHUNK = 128
TB = 80


def _tile_rows(n):
    rpt = (n // NS) // 8 * 8
    rem = n - rpt * NS
    return rpt, rem


def _hist_body(nchunks, src2, dst2, zeros_h, row_ids, out,
               sidx, didx, hs, hd, ridx, sh_s, sh_d):
    nrows = sh_s.shape[0]
    c = lax.axis_index("c")
    s = lax.axis_index("s")
    wid = s * NC + c
    nw8 = nrows // 8
    nft = nchunks // TB
    last = nchunks - nft * TB
    base = wid * TB

    pltpu.sync_copy(zeros_h, hs)
    pltpu.sync_copy(zeros_h, hd)

    @pl.when(s < nw8)
    def _():
        pltpu.sync_copy(zeros_h.at[pl.ds(0, 8)], sh_s.at[pl.ds(s * 8, 8)])
        pltpu.sync_copy(zeros_h.at[pl.ds(0, 8)], sh_d.at[pl.ds(s * 8, 8)])

    pltpu.sync_copy(row_ids, ridx)

    @pl.when(wid < nft)
    def _():
        pltpu.sync_copy(src2.at[pl.ds(base, TB)], sidx.at[pl.ds(0, TB)])
        pltpu.sync_copy(dst2.at[pl.ds(base, TB)], didx.at[pl.ds(0, TB)])
    if last:
        @pl.when(wid == nft)
        def _():
            pltpu.sync_copy(src2.at[pl.ds(base, last)], sidx.at[pl.ds(0, last)])
            pltpu.sync_copy(dst2.at[pl.ds(base, last)], didx.at[pl.ds(0, last)])

    plsc.subcore_barrier()

    ones16 = jnp.ones((16,), jnp.float32)

    def scat(hist, iv):
        plsc.addupdate_scatter(
            hist, [lax.shift_right_logical(iv, 7), lax.bitwise_and(iv, 127)],
            ones16)

    cnt = jnp.maximum(0, jnp.minimum(TB, nchunks - base))

    def body(a, carry):
        for j in range(CHUNK // 16):
            scat(hs, sidx[a, pl.ds(j * 16, 16)])
            scat(hd, didx[a, pl.ds(j * 16, 16)])
        return carry

    lax.fori_loop(0, cnt, body, 0)

    pltpu.sync_copy(hs, sh_s.at[ridx], add=True)
    pltpu.sync_copy(hd, sh_d.at[ridx], add=True)
    plsc.subcore_barrier()

    @pl.when(s < nw8)
    def _():
        pltpu.sync_copy(sh_s.at[pl.ds(s * 8, 8)], out.at[c, 0, pl.ds(s * 8, 8)])
        pltpu.sync_copy(sh_d.at[pl.ds(s * 8, 8)], out.at[c, 1, pl.ds(s * 8, 8)])


def _sc_hist(src2, dst2, n):
    nchunks = src2.shape[0]
    nrows = -(-n // (128 * NS)) * NS
    zeros_h = jnp.zeros((nrows, 128), jnp.float32)
    row_ids = jnp.arange(nrows, dtype=jnp.int32)
    mesh = plsc.VectorSubcoreMesh(core_axis_name="c", subcore_axis_name="s")
    hist = pl.kernel(
        functools.partial(_hist_body, nchunks),
        out_type=jax.ShapeDtypeStruct((NC, 2, nrows, 128), jnp.float32),
        mesh=mesh,
        scratch_types=[
            pltpu.VMEM((TB, CHUNK), jnp.int32),
            pltpu.VMEM((TB, CHUNK), jnp.int32),
            pltpu.VMEM((nrows, 128), jnp.float32),
            pltpu.VMEM((nrows, 128), jnp.float32),
            pltpu.VMEM((nrows,), jnp.int32),
            pltpu.VMEM_SHARED((nrows, 128), jnp.float32),
            pltpu.VMEM_SHARED((nrows, 128), jnp.float32),
        ],
        compiler_params=pltpu.CompilerParams(needs_layout_passes=False),
    )(src2, dst2, zeros_h, row_ids)
    return hist.reshape(NC, 2, nrows * 128)[:, :, :n, None]


def _agg_body(nchunks, src2, dst2, y2, zrows, out,
              sidx, didx, rows_a, rows_b, agg, gsem, ssem):
    n, d = agg.shape
    c = lax.axis_index("c")
    s = lax.axis_index("s")
    wid = s * NC + c
    rpt, rem = _tile_rows(n)
    nft = nchunks // TB
    last = nchunks - nft * TB
    base = wid * TB

    pltpu.sync_copy(zrows.at[pl.ds(0, rpt)], agg.at[pl.ds(s * rpt, rpt)])

    @pl.when(s == NS - 1)
    def _():
        pltpu.sync_copy(zrows.at[pl.ds(0, rem)], agg.at[pl.ds(NS * rpt, rem)])

    plsc.subcore_barrier()

    def gather(j, buf):
        return pltpu.make_async_copy(y2.at[sidx.at[j]], buf, gsem)

    def scatter(j, buf):
        return pltpu.make_async_copy(buf, agg.at[didx.at[j]], ssem)

    cnt = jnp.maximum(0, jnp.minimum(TB, nchunks - base))
    nblk = cnt // 8

    def block(m, carry):
        bbase = base + m * 8
        pltpu.sync_copy(src2.at[pl.ds(bbase, 8)], sidx)
        pltpu.sync_copy(dst2.at[pl.ds(bbase, 8)], didx)
        gather(0, rows_a).start()
        for k in range(4):
            a = 2 * k
            b = a + 1
            gather(a, rows_a).wait()
            sc_a = scatter(a, rows_a)
            sc_a.start(add=True)
            g_b = gather(b, rows_b)
            g_b.start()
            g_b.wait()
            sc_a.wait()
            sc_b = scatter(b, rows_b)
            sc_b.start(add=True)
            if k < 3:
                gather(a + 2, rows_a).start()
            sc_b.wait()
        return carry

    lax.fori_loop(0, nblk, block, 0)

    plsc.subcore_barrier()

    pltpu.sync_copy(agg.at[pl.ds(s * rpt, rpt)],
                    out.at[c, pl.ds(s * rpt, rpt)])

    @pl.when(s == NS - 1)
    def _():
        pltpu.sync_copy(agg.at[pl.ds(NS * rpt, rem)],
                        out.at[c, pl.ds(NS * rpt, rem)])


def _sc_aggregate(src2, dst2, y2, n_out):
    nchunks = src2.shape[0]
    d = y2.shape[1]
    rpt, _ = _tile_rows(n_out)
    zrows = jnp.zeros((rpt, d), jnp.float32)
    mesh = plsc.VectorSubcoreMesh(core_axis_name="c", subcore_axis_name="s")
    return pl.kernel(
        functools.partial(_agg_body, nchunks),
        out_type=jax.ShapeDtypeStruct((NC, n_out, d), jnp.float32),
        mesh=mesh,
        scratch_types=[
            pltpu.VMEM((8, CHUNK), jnp.int32),
            pltpu.VMEM((8, CHUNK), jnp.int32),
            pltpu.VMEM((CHUNK, d), jnp.float32),
            pltpu.VMEM((CHUNK, d), jnp.float32),
            pltpu.VMEM_SHARED((n_out, d), jnp.float32),
            pltpu.SemaphoreType.DMA,
            pltpu.SemaphoreType.DMA,
        ],
    )(src2, dst2, y2, zrows)


def _dinv(deg):
    return jnp.where(deg > 0, lax.rsqrt(jnp.maximum(deg, 1e-12)), 0.0)


def _matmul_scale_body(x_ref, w_ref, hist_ref, y2_ref):
    deg_src = hist_ref[0, 0] + hist_ref[1, 0]
    y = jnp.dot(x_ref[...], w_ref[...], preferred_element_type=jnp.float32)
    y2_ref[...] = y * _dinv(deg_src)


def _tc_matmul_scale(x, w, hist, blk=2000):
    n, dout = x.shape[0], w.shape[1]
    grid = n // blk
    return pl.pallas_call(
        _matmul_scale_body,
        grid=(grid,),
        in_specs=[
            pl.BlockSpec((blk, x.shape[1]), lambda i: (i, 0)),
            pl.BlockSpec((w.shape[0], dout), lambda i: (0, 0)),
            pl.BlockSpec((NC, 2, blk, 1), lambda i: (0, 0, i, 0)),
        ],
        out_specs=pl.BlockSpec((blk, dout), lambda i: (i, 0)),
        out_shape=jax.ShapeDtypeStruct((n, dout), jnp.float32),
    )(x, w, hist)


def _final_body(parts_ref, hist_ref, b_ref, out_ref):
    deg_dst = hist_ref[0, 1] + hist_ref[1, 1]
    agg = parts_ref[0] + parts_ref[1]
    out_ref[...] = agg * _dinv(deg_dst) + b_ref[...]


def _tc_final(parts, hist, b, blk=2000):
    n, d = parts.shape[1], parts.shape[2]
    grid = n // blk
    return pl.pallas_call(
        _final_body,
        grid=(grid,),
        in_specs=[
            pl.BlockSpec((NC, blk, d), lambda i: (0, i, 0)),
            pl.BlockSpec((NC, 2, blk, 1), lambda i: (0, 0, i, 0)),
            pl.BlockSpec((1, d), lambda i: (0, 0)),
        ],
        out_specs=pl.BlockSpec((blk, d), lambda i: (i, 0)),
        out_shape=jax.ShapeDtypeStruct((n, d), jnp.float32),
    )(parts, hist, b.reshape(1, d))


@jax.jit
def kernel(x, edge_index, W, b):
    n = x.shape[0]
    e = edge_index.shape[1]
    e_pad = -(-e // (8 * CHUNK)) * (8 * CHUNK)
    src_e, dst_e = edge_index[0], edge_index[1]
    dst_agg = dst_e
    if e_pad > e:
        fill_n = jnp.full((e_pad - e,), n, dtype=src_e.dtype)
        fill_0 = jnp.zeros((e_pad - e,), dtype=src_e.dtype)
        src_e = jnp.concatenate([src_e, fill_n])
        dst_e = jnp.concatenate([dst_e, fill_n])
        dst_agg = jnp.concatenate([dst_agg, fill_0])
    src2 = src_e.reshape(e_pad // CHUNK, CHUNK)
    dst2 = dst_e.reshape(e_pad // CHUNK, CHUNK)
    dst2a = dst_agg.reshape(e_pad // CHUNK, CHUNK)

    hist = _sc_hist(src2, dst2, n)
    y2 = _tc_matmul_scale(x, W, hist)
    if e_pad > e:
        y2 = jnp.pad(y2, ((0, 8), (0, 0)))
    parts = _sc_aggregate(src2, dst2a, y2, n)
    return _tc_final(parts, hist, b)


# speedup vs baseline: 29.2478x; 1.0387x over previous
import functools

import jax
import jax.numpy as jnp
from jax import lax
from jax.experimental import pallas as pl
from jax.experimental.pallas import tpu as pltpu
from jax.experimental.pallas import tpu_sc as plsc

NC = 2
NS = 16
NW = NC * NS
CHUNK = 128
TB = 80


def _tile_rows(n):
    rpt = (n // NS) // 8 * 8
    rem = n - rpt * NS
    return rpt, rem


def _hist_body(nchunks, src2, dst2, zeros_h, row_ids, out,
               sidx, didx, hs, hd, ridx, sh_s, sh_d):
    nrows = sh_s.shape[0]
    c = lax.axis_index("c")
    s = lax.axis_index("s")
    wid = s * NC + c
    nw8 = nrows // 8
    nft = nchunks // TB
    last = nchunks - nft * TB
    base = wid * TB

    pltpu.sync_copy(zeros_h, hs)
    pltpu.sync_copy(zeros_h, hd)

    @pl.when(s < nw8)
    def _():
        pltpu.sync_copy(zeros_h.at[pl.ds(0, 8)], sh_s.at[pl.ds(s * 8, 8)])
        pltpu.sync_copy(zeros_h.at[pl.ds(0, 8)], sh_d.at[pl.ds(s * 8, 8)])

    pltpu.sync_copy(row_ids, ridx)

    @pl.when(wid < nft)
    def _():
        pltpu.sync_copy(src2.at[pl.ds(base, TB)], sidx.at[pl.ds(0, TB)])
        pltpu.sync_copy(dst2.at[pl.ds(base, TB)], didx.at[pl.ds(0, TB)])
    if last:
        @pl.when(wid == nft)
        def _():
            pltpu.sync_copy(src2.at[pl.ds(base, last)], sidx.at[pl.ds(0, last)])
            pltpu.sync_copy(dst2.at[pl.ds(base, last)], didx.at[pl.ds(0, last)])

    plsc.subcore_barrier()

    ones16 = jnp.ones((16,), jnp.float32)

    def scat(hist, iv):
        plsc.addupdate_scatter(
            hist, [lax.shift_right_logical(iv, 7), lax.bitwise_and(iv, 127)],
            ones16)

    cnt = jnp.maximum(0, jnp.minimum(TB, nchunks - base))

    def body(a, carry):
        for j in range(CHUNK // 16):
            scat(hs, sidx[a, pl.ds(j * 16, 16)])
            scat(hd, didx[a, pl.ds(j * 16, 16)])
        return carry

    lax.fori_loop(0, cnt, body, 0)

    pltpu.sync_copy(hs, sh_s.at[ridx], add=True)
    pltpu.sync_copy(hd, sh_d.at[ridx], add=True)
    plsc.subcore_barrier()

    @pl.when(s < nw8)
    def _():
        pltpu.sync_copy(sh_s.at[pl.ds(s * 8, 8)], out.at[c, 0, pl.ds(s * 8, 8)])
        pltpu.sync_copy(sh_d.at[pl.ds(s * 8, 8)], out.at[c, 1, pl.ds(s * 8, 8)])


def _sc_hist(src2, dst2, n):
    nchunks = src2.shape[0]
    nrows = -(-n // (128 * NS)) * NS
    zeros_h = jnp.zeros((nrows, 128), jnp.float32)
    row_ids = jnp.arange(nrows, dtype=jnp.int32)
    mesh = plsc.VectorSubcoreMesh(core_axis_name="c", subcore_axis_name="s")
    hist = pl.kernel(
        functools.partial(_hist_body, nchunks),
        out_type=jax.ShapeDtypeStruct((NC, 2, nrows, 128), jnp.float32),
        mesh=mesh,
        scratch_types=[
            pltpu.VMEM((TB, CHUNK), jnp.int32),
            pltpu.VMEM((TB, CHUNK), jnp.int32),
            pltpu.VMEM((nrows, 128), jnp.float32),
            pltpu.VMEM((nrows, 128), jnp.float32),
            pltpu.VMEM((nrows,), jnp.int32),
            pltpu.VMEM_SHARED((nrows, 128), jnp.float32),
            pltpu.VMEM_SHARED((nrows, 128), jnp.float32),
        ],
        compiler_params=pltpu.CompilerParams(needs_layout_passes=False),
    )(src2, dst2, zeros_h, row_ids)
    return hist.reshape(NC, 2, nrows * 128)[:, :, :n, None]


def _agg_body(nchunks, tba, comb2, y2, zrows, out,
              cidx, srow_a, drow_a, srow_b, drow_b, rows_a, rows_b,
              agg, gsem, ssem):
    n, d = agg.shape
    c = lax.axis_index("c")
    s = lax.axis_index("s")
    wid = s * NC + c
    rpt, rem = _tile_rows(n)
    nft = nchunks // tba
    last = nchunks - nft * tba
    base = wid * tba

    pltpu.sync_copy(zrows.at[pl.ds(0, rpt)], agg.at[pl.ds(s * rpt, rpt)])

    @pl.when(s == NS - 1)
    def _():
        pltpu.sync_copy(zrows.at[pl.ds(0, rem)], agg.at[pl.ds(NS * rpt, rem)])

    @pl.when(wid < nft)
    def _():
        pltpu.sync_copy(comb2.at[pl.ds(base, tba)], cidx.at[pl.ds(0, tba)])
    if last:
        @pl.when(wid == nft)
        def _():
            pltpu.sync_copy(comb2.at[pl.ds(base, last)], cidx.at[pl.ds(0, last)])

    plsc.subcore_barrier()

    def unpack(j, srow, drow):
        for i in range(CHUNK // 16):
            v = cidx[j, pl.ds(i * 16, 16)]
            srow[pl.ds(i * 16, 16)] = lax.bitwise_and(v, 0xFFFF)
            drow[pl.ds(i * 16, 16)] = lax.shift_right_logical(v, 16)

    def gather(srow, buf):
        return pltpu.make_async_copy(y2.at[srow], buf, gsem)

    def scatter(buf, drow):
        return pltpu.make_async_copy(buf, agg.at[drow], ssem)

    cnt = jnp.maximum(0, jnp.minimum(tba, nchunks - base))
    half = cnt // 2

    @pl.when(half > 0)
    def _():
        unpack(0, srow_a, drow_a)
        gather(srow_a, rows_a).start()

        def body(k, carry):
            a = 2 * k
            b = a + 1
            unpack(b, srow_b, drow_b)
            gather(srow_a, rows_a).wait()
            sc_a = scatter(rows_a, drow_a)
            sc_a.start(add=True)
            g_b = gather(srow_b, rows_b)
            g_b.start()
            g_b.wait()
            sc_a.wait()
            sc_b = scatter(rows_b, drow_b)
            sc_b.start(add=True)

            @pl.when(k + 1 < half)
            def _():
                unpack(a + 2, srow_a, drow_a)
                gather(srow_a, rows_a).start()

            sc_b.wait()
            return carry

        lax.fori_loop(0, half, body, 0)

    plsc.subcore_barrier()

    pltpu.sync_copy(agg.at[pl.ds(s * rpt, rpt)],
                    out.at[c, pl.ds(s * rpt, rpt)])

    @pl.when(s == NS - 1)
    def _():
        pltpu.sync_copy(agg.at[pl.ds(NS * rpt, rem)],
                        out.at[c, pl.ds(NS * rpt, rem)])


def _sc_aggregate(comb2, y2, n_out):
    nchunks = comb2.shape[0]
    d = y2.shape[1]
    rpt, _ = _tile_rows(n_out)
    tba = -(-(-(-nchunks // NW)) // 8) * 8
    zrows = jnp.zeros((rpt, d), jnp.float32)
    mesh = plsc.VectorSubcoreMesh(core_axis_name="c", subcore_axis_name="s")
    return pl.kernel(
        functools.partial(_agg_body, nchunks, tba),
        out_type=jax.ShapeDtypeStruct((NC, n_out, d), jnp.float32),
        mesh=mesh,
        scratch_types=[
            pltpu.VMEM((tba, CHUNK), jnp.int32),
            pltpu.VMEM((CHUNK,), jnp.int32),
            pltpu.VMEM((CHUNK,), jnp.int32),
            pltpu.VMEM((CHUNK,), jnp.int32),
            pltpu.VMEM((CHUNK,), jnp.int32),
            pltpu.VMEM((CHUNK, d), jnp.float32),
            pltpu.VMEM((CHUNK, d), jnp.float32),
            pltpu.VMEM_SHARED((n_out, d), jnp.float32),
            pltpu.SemaphoreType.DMA,
            pltpu.SemaphoreType.DMA,
        ],
        compiler_params=pltpu.CompilerParams(needs_layout_passes=False),
    )(comb2, y2, zrows)


def _dinv(deg):
    return jnp.where(deg > 0, lax.rsqrt(jnp.maximum(deg, 1e-12)), 0.0)


def _matmul_scale_body(x_ref, w_ref, hist_ref, y2_ref):
    deg_src = hist_ref[0, 0] + hist_ref[1, 0]
    y = jnp.dot(x_ref[...], w_ref[...], preferred_element_type=jnp.float32)
    y2_ref[...] = y * _dinv(deg_src)


def _tc_matmul_scale(x, w, hist, blk=2000):
    n, dout = x.shape[0], w.shape[1]
    grid = n // blk
    return pl.pallas_call(
        _matmul_scale_body,
        grid=(grid,),
        in_specs=[
            pl.BlockSpec((blk, x.shape[1]), lambda i: (i, 0)),
            pl.BlockSpec((w.shape[0], dout), lambda i: (0, 0)),
            pl.BlockSpec((NC, 2, blk, 1), lambda i: (0, 0, i, 0)),
        ],
        out_specs=pl.BlockSpec((blk, dout), lambda i: (i, 0)),
        out_shape=jax.ShapeDtypeStruct((n, dout), jnp.float32),
    )(x, w, hist)


def _final_body(parts_ref, hist_ref, b_ref, out_ref):
    deg_dst = hist_ref[0, 1] + hist_ref[1, 1]
    agg = parts_ref[0] + parts_ref[1]
    out_ref[...] = agg * _dinv(deg_dst) + b_ref[...]


def _tc_final(parts, hist, b, blk=2000):
    n, d = parts.shape[1], parts.shape[2]
    grid = n // blk
    return pl.pallas_call(
        _final_body,
        grid=(grid,),
        in_specs=[
            pl.BlockSpec((NC, blk, d), lambda i: (0, i, 0)),
            pl.BlockSpec((NC, 2, blk, 1), lambda i: (0, 0, i, 0)),
            pl.BlockSpec((1, d), lambda i: (0, 0)),
        ],
        out_specs=pl.BlockSpec((blk, d), lambda i: (i, 0)),
        out_shape=jax.ShapeDtypeStruct((n, d), jnp.float32),
    )(parts, hist, b.reshape(1, d))


@jax.jit
def kernel(x, edge_index, W, b):
    n = x.shape[0]
    e = edge_index.shape[1]
    e_pad = -(-e // (8 * CHUNK)) * (8 * CHUNK)
    src_e, dst_e = edge_index[0], edge_index[1]
    dst_agg = dst_e
    if e_pad > e:
        fill_n = jnp.full((e_pad - e,), n, dtype=src_e.dtype)
        fill_0 = jnp.zeros((e_pad - e,), dtype=src_e.dtype)
        src_e = jnp.concatenate([src_e, fill_n])
        dst_e = jnp.concatenate([dst_e, fill_n])
        dst_agg = jnp.concatenate([dst_agg, fill_0])
    src2 = src_e.reshape(e_pad // CHUNK, CHUNK)
    dst2 = dst_e.reshape(e_pad // CHUNK, CHUNK)
    comb2 = (src_e | (dst_agg << 16)).reshape(e_pad // CHUNK, CHUNK)

    hist = _sc_hist(src2, dst2, n)
    y2 = _tc_matmul_scale(x, W, hist)
    if e_pad > e:
        y2 = jnp.pad(y2, ((0, 8), (0, 0)))
    parts = _sc_aggregate(comb2, y2, n)
    return _tc_final(parts, hist, b)


# speedup vs baseline: 29.5177x; 1.0092x over previous
import functools

import jax
import jax.numpy as jnp
from jax import lax
from jax.experimental import pallas as pl
from jax.experimental.pallas import tpu as pltpu
from jax.experimental.pallas import tpu_sc as plsc

NC = 2
NS = 16
NW = NC * NS
CHUNK = 128
TB = 80


def _tile_rows(n):
    rpt = (n // NS) // 8 * 8
    rem = n - rpt * NS
    return rpt, rem


def _hist_body(nchunks, src2, dst2, zeros_h, row_ids, out,
               sidx, didx, hs, hd, ridx, sh_s, sh_d):
    nrows = sh_s.shape[0]
    c = lax.axis_index("c")
    s = lax.axis_index("s")
    wid = s * NC + c
    nw8 = nrows // 8
    nft = nchunks // TB
    last = nchunks - nft * TB
    base = wid * TB

    pltpu.sync_copy(zeros_h, hs)
    pltpu.sync_copy(zeros_h, hd)

    @pl.when(s < nw8)
    def _():
        pltpu.sync_copy(zeros_h.at[pl.ds(0, 8)], sh_s.at[pl.ds(s * 8, 8)])
        pltpu.sync_copy(zeros_h.at[pl.ds(0, 8)], sh_d.at[pl.ds(s * 8, 8)])

    pltpu.sync_copy(row_ids, ridx)

    @pl.when(wid < nft)
    def _():
        pltpu.sync_copy(src2.at[pl.ds(base, TB)], sidx.at[pl.ds(0, TB)])
        pltpu.sync_copy(dst2.at[pl.ds(base, TB)], didx.at[pl.ds(0, TB)])
    if last:
        @pl.when(wid == nft)
        def _():
            pltpu.sync_copy(src2.at[pl.ds(base, last)], sidx.at[pl.ds(0, last)])
            pltpu.sync_copy(dst2.at[pl.ds(base, last)], didx.at[pl.ds(0, last)])

    plsc.subcore_barrier()

    ones16 = jnp.ones((16,), jnp.float32)

    def scat(hist, iv):
        plsc.addupdate_scatter(
            hist, [lax.shift_right_logical(iv, 7), lax.bitwise_and(iv, 127)],
            ones16)

    cnt = jnp.maximum(0, jnp.minimum(TB, nchunks - base))

    def body(a, carry):
        for j in range(CHUNK // 16):
            scat(hs, sidx[a, pl.ds(j * 16, 16)])
            scat(hd, didx[a, pl.ds(j * 16, 16)])
        return carry

    lax.fori_loop(0, cnt, body, 0)

    pltpu.sync_copy(hs, sh_s.at[ridx], add=True)
    pltpu.sync_copy(hd, sh_d.at[ridx], add=True)
    plsc.subcore_barrier()

    @pl.when(s < nw8)
    def _():
        pltpu.sync_copy(sh_s.at[pl.ds(s * 8, 8)], out.at[c, 0, pl.ds(s * 8, 8)])
        pltpu.sync_copy(sh_d.at[pl.ds(s * 8, 8)], out.at[c, 1, pl.ds(s * 8, 8)])


def _sc_hist(src2, dst2, n):
    nchunks = src2.shape[0]
    nrows = -(-n // (128 * NS)) * NS
    zeros_h = jnp.zeros((nrows, 128), jnp.float32)
    row_ids = jnp.arange(nrows, dtype=jnp.int32)
    mesh = plsc.VectorSubcoreMesh(core_axis_name="c", subcore_axis_name="s")
    hist = pl.kernel(
        functools.partial(_hist_body, nchunks),
        out_type=jax.ShapeDtypeStruct((NC, 2, nrows, 128), jnp.float32),
        mesh=mesh,
        scratch_types=[
            pltpu.VMEM((TB, CHUNK), jnp.int32),
            pltpu.VMEM((TB, CHUNK), jnp.int32),
            pltpu.VMEM((nrows, 128), jnp.float32),
            pltpu.VMEM((nrows, 128), jnp.float32),
            pltpu.VMEM((nrows,), jnp.int32),
            pltpu.VMEM_SHARED((nrows, 128), jnp.float32),
            pltpu.VMEM_SHARED((nrows, 128), jnp.float32),
        ],
        compiler_params=pltpu.CompilerParams(needs_layout_passes=False),
    )(src2, dst2, zeros_h, row_ids)
    return hist.reshape(NC, 2, nrows * 128)[:, :, :n, None]


def _agg_body(nchunks, tba, comb2, y2, zrows, out,
              cidx, srow_a, drow_a, srow_b, drow_b, rows_a, rows_b,
              agg, gsem, ssem):
    n, d = agg.shape
    c = lax.axis_index("c")
    s = lax.axis_index("s")
    wid = s * NC + c
    rpt, rem = _tile_rows(n)
    nft = nchunks // tba
    last = nchunks - nft * tba
    base = wid * tba

    pltpu.sync_copy(zrows.at[pl.ds(0, rpt)], agg.at[pl.ds(s * rpt, rpt)])

    @pl.when(s == NS - 1)
    def _():
        pltpu.sync_copy(zrows.at[pl.ds(0, rem)], agg.at[pl.ds(NS * rpt, rem)])

    @pl.when(wid < nft)
    def _():
        pltpu.sync_copy(comb2.at[pl.ds(base, tba)], cidx.at[pl.ds(0, tba)])
    if last:
        @pl.when(wid == nft)
        def _():
            pltpu.sync_copy(comb2.at[pl.ds(base, last)], cidx.at[pl.ds(0, last)])

    plsc.subcore_barrier()

    def unpack(j, srow, drow):
        for i in range(CHUNK // 16):
            v = cidx[j, pl.ds(i * 16, 16)]
            srow[pl.ds(i * 16, 16)] = lax.bitwise_and(v, 0xFFFF)
            drow[pl.ds(i * 16, 16)] = lax.shift_right_logical(v, 16)

    def gather(srow, buf):
        return pltpu.make_async_copy(y2.at[srow], buf, gsem)

    def scatter(buf, drow):
        return pltpu.make_async_copy(buf, agg.at[drow], ssem)

    cnt = jnp.maximum(0, jnp.minimum(tba, nchunks - base))
    half = cnt // 2

    @pl.when(half > 0)
    def _():
        unpack(0, srow_a, drow_a)
        gather(srow_a, rows_a).start()

        def body(k, carry):
            a = 2 * k
            b = a + 1
            unpack(b, srow_b, drow_b)
            gather(srow_a, rows_a).wait()
            sc_a = scatter(rows_a, drow_a)
            sc_a.start(add=True)
            g_b = gather(srow_b, rows_b)
            g_b.start()
            g_b.wait()
            sc_a.wait()
            sc_b = scatter(rows_b, drow_b)
            sc_b.start(add=True)

            @pl.when(k + 1 < half)
            def _():
                unpack(a + 2, srow_a, drow_a)
                gather(srow_a, rows_a).start()

            sc_b.wait()
            return carry

        lax.fori_loop(0, half, body, 0)

    plsc.subcore_barrier()

    pltpu.sync_copy(agg.at[pl.ds(s * rpt, rpt)],
                    out.at[c, pl.ds(s * rpt, rpt)])

    @pl.when(s == NS - 1)
    def _():
        pltpu.sync_copy(agg.at[pl.ds(NS * rpt, rem)],
                        out.at[c, pl.ds(NS * rpt, rem)])


def _sc_aggregate(comb2, y2, n_out):
    nchunks = comb2.shape[0]
    d = y2.shape[1]
    rpt, _ = _tile_rows(n_out)
    tba = -(-(-(-nchunks // NW)) // 8) * 8
    zrows = jnp.zeros((rpt, d), jnp.float32)
    mesh = plsc.VectorSubcoreMesh(core_axis_name="c", subcore_axis_name="s")
    return pl.kernel(
        functools.partial(_agg_body, nchunks, tba),
        out_type=jax.ShapeDtypeStruct((NC, n_out, d), jnp.float32),
        mesh=mesh,
        scratch_types=[
            pltpu.VMEM((tba, CHUNK), jnp.int32),
            pltpu.VMEM((CHUNK,), jnp.int32),
            pltpu.VMEM((CHUNK,), jnp.int32),
            pltpu.VMEM((CHUNK,), jnp.int32),
            pltpu.VMEM((CHUNK,), jnp.int32),
            pltpu.VMEM((CHUNK, d), jnp.float32),
            pltpu.VMEM((CHUNK, d), jnp.float32),
            pltpu.VMEM_SHARED((n_out, d), jnp.float32),
            pltpu.SemaphoreType.DMA,
            pltpu.SemaphoreType.DMA,
        ],
        compiler_params=pltpu.CompilerParams(needs_layout_passes=False),
    )(comb2, y2, zrows)


def _dinv(deg):
    return jnp.where(deg > 0, lax.rsqrt(jnp.maximum(deg, 1e-12)), 0.0)


def _matmul_body(x_ref, w_ref, y_ref):
    y_ref[...] = jnp.dot(x_ref[...], w_ref[...],
                         preferred_element_type=jnp.float32)


def _tc_matmul(x, w, n_pad, blk=2000):
    dout = w.shape[1]
    grid = -(-n_pad // blk)
    return pl.pallas_call(
        _matmul_body,
        grid=(grid,),
        in_specs=[
            pl.BlockSpec((blk, x.shape[1]), lambda i: (i, 0)),
            pl.BlockSpec((w.shape[0], dout), lambda i: (0, 0)),
        ],
        out_specs=pl.BlockSpec((blk, dout), lambda i: (i, 0)),
        out_shape=jax.ShapeDtypeStruct((n_pad, dout), jnp.float32),
    )(x, w)


def _scale_body(nv, blk, y_ref, hist_ref, y2_ref):
    i = pl.program_id(0)
    deg_src = hist_ref[0, 0] + hist_ref[1, 0]
    y2 = y_ref[...] * _dinv(deg_src)
    rows = i * blk + lax.broadcasted_iota(jnp.int32, y2.shape, 0)
    y2_ref[...] = jnp.where(rows < nv, y2, 0.0)


def _tc_scale(y, hist, n, blk=2000):
    n_pad, d = y.shape
    grid = -(-n_pad // blk)
    return pl.pallas_call(
        functools.partial(_scale_body, n, blk),
        grid=(grid,),
        in_specs=[
            pl.BlockSpec((blk, d), lambda i: (i, 0)),
            pl.BlockSpec((NC, 2, blk, 1), lambda i: (0, 0, i, 0)),
        ],
        out_specs=pl.BlockSpec((blk, d), lambda i: (i, 0)),
        out_shape=jax.ShapeDtypeStruct((n_pad, d), jnp.float32),
        input_output_aliases={0: 0},
    )(y, hist)


def _final_body(parts_ref, hist_ref, b_ref, out_ref):
    deg_dst = hist_ref[0, 1] + hist_ref[1, 1]
    agg = parts_ref[0] + parts_ref[1]
    out_ref[...] = agg * _dinv(deg_dst) + b_ref[...]


def _tc_final(parts, hist, b, blk=2000):
    n, d = parts.shape[1], parts.shape[2]
    grid = n // blk
    return pl.pallas_call(
        _final_body,
        grid=(grid,),
        in_specs=[
            pl.BlockSpec((NC, blk, d), lambda i: (0, i, 0)),
            pl.BlockSpec((NC, 2, blk, 1), lambda i: (0, 0, i, 0)),
            pl.BlockSpec((1, d), lambda i: (0, 0)),
        ],
        out_specs=pl.BlockSpec((blk, d), lambda i: (i, 0)),
        out_shape=jax.ShapeDtypeStruct((n, d), jnp.float32),
    )(parts, hist, b.reshape(1, d))


@jax.jit
def kernel(x, edge_index, W, b):
    n = x.shape[0]
    e = edge_index.shape[1]
    e_pad = -(-e // (8 * CHUNK)) * (8 * CHUNK)
    src_e, dst_e = edge_index[0], edge_index[1]
    dst_agg = dst_e
    if e_pad > e:
        fill_n = jnp.full((e_pad - e,), n, dtype=src_e.dtype)
        fill_0 = jnp.zeros((e_pad - e,), dtype=src_e.dtype)
        src_e = jnp.concatenate([src_e, fill_n])
        dst_e = jnp.concatenate([dst_e, fill_n])
        dst_agg = jnp.concatenate([dst_agg, fill_0])
    src2 = src_e.reshape(e_pad // CHUNK, CHUNK)
    dst2 = dst_e.reshape(e_pad // CHUNK, CHUNK)
    comb2 = (src_e | (dst_agg << 16)).reshape(e_pad // CHUNK, CHUNK)

    n_pad = n + 8 if e_pad > e else n
    hist = _sc_hist(src2, dst2, n)
    y = _tc_matmul(x, W, n_pad)
    y2 = _tc_scale(y, hist, n)
    parts = _sc_aggregate(comb2, y2, n)
    return _tc_final(parts, hist, b)


# speedup vs baseline: 29.6429x; 1.0042x over previous
import functools

import jax
import jax.numpy as jnp
from jax import lax
from jax.experimental import pallas as pl
from jax.experimental.pallas import tpu as pltpu
from jax.experimental.pallas import tpu_sc as plsc

NC = 2
NS = 16
NW = NC * NS
CHUNK = 128
TB = 80


def _tile_rows(n):
    rpt = (n // NS) // 8 * 8
    rem = n - rpt * NS
    return rpt, rem


def _hist_body(nchunks, comb2, zeros_h, row_ids, out,
               cidx, hs, hd, ridx, sh_s, sh_d):
    nrows = sh_s.shape[0]
    c = lax.axis_index("c")
    s = lax.axis_index("s")
    wid = s * NC + c
    nw8 = nrows // 8
    nft = nchunks // TB
    last = nchunks - nft * TB
    base = wid * TB

    pltpu.sync_copy(zeros_h, hs)
    pltpu.sync_copy(zeros_h, hd)

    @pl.when(s < nw8)
    def _():
        pltpu.sync_copy(zeros_h.at[pl.ds(0, 8)], sh_s.at[pl.ds(s * 8, 8)])
        pltpu.sync_copy(zeros_h.at[pl.ds(0, 8)], sh_d.at[pl.ds(s * 8, 8)])

    pltpu.sync_copy(row_ids, ridx)

    @pl.when(wid < nft)
    def _():
        pltpu.sync_copy(comb2.at[pl.ds(base, TB)], cidx.at[pl.ds(0, TB)])
    if last:
        @pl.when(wid == nft)
        def _():
            pltpu.sync_copy(comb2.at[pl.ds(base, last)], cidx.at[pl.ds(0, last)])

    plsc.subcore_barrier()

    ones16 = jnp.ones((16,), jnp.float32)

    def scat(hist, iv):
        plsc.addupdate_scatter(
            hist, [lax.shift_right_logical(iv, 7), lax.bitwise_and(iv, 127)],
            ones16)

    cnt = jnp.maximum(0, jnp.minimum(TB, nchunks - base))

    def body(a, carry):
        for j in range(CHUNK // 16):
            v = cidx[a, pl.ds(j * 16, 16)]
            scat(hs, lax.bitwise_and(v, 0xFFFF))
            scat(hd, lax.shift_right_logical(v, 16))
        return carry

    lax.fori_loop(0, cnt, body, 0)

    pltpu.sync_copy(hs, sh_s.at[ridx], add=True)
    pltpu.sync_copy(hd, sh_d.at[ridx], add=True)
    plsc.subcore_barrier()

    @pl.when(s < nw8)
    def _():
        pltpu.sync_copy(sh_s.at[pl.ds(s * 8, 8)], out.at[c, 0, pl.ds(s * 8, 8)])
        pltpu.sync_copy(sh_d.at[pl.ds(s * 8, 8)], out.at[c, 1, pl.ds(s * 8, 8)])


def _sc_hist(comb2, n):
    nchunks = comb2.shape[0]
    nrows = -(-n // (128 * NS)) * NS
    zeros_h = jnp.zeros((nrows, 128), jnp.float32)
    row_ids = jnp.arange(nrows, dtype=jnp.int32)
    mesh = plsc.VectorSubcoreMesh(core_axis_name="c", subcore_axis_name="s")
    hist = pl.kernel(
        functools.partial(_hist_body, nchunks),
        out_type=jax.ShapeDtypeStruct((NC, 2, nrows, 128), jnp.float32),
        mesh=mesh,
        scratch_types=[
            pltpu.VMEM((TB, CHUNK), jnp.int32),
            pltpu.VMEM((nrows, 128), jnp.float32),
            pltpu.VMEM((nrows, 128), jnp.float32),
            pltpu.VMEM((nrows,), jnp.int32),
            pltpu.VMEM_SHARED((nrows, 128), jnp.float32),
            pltpu.VMEM_SHARED((nrows, 128), jnp.float32),
        ],
        compiler_params=pltpu.CompilerParams(needs_layout_passes=False),
    )(comb2, zeros_h, row_ids)
    return hist.reshape(NC, 2, nrows * 128)[:, :, :n, None]


def _agg_body(nchunks, tba, comb2, y2, zrows, out,
              cidx, srow_a, drow_a, srow_b, drow_b, rows_a, rows_b,
              agg, gsem, ssem):
    n, d = agg.shape
    c = lax.axis_index("c")
    s = lax.axis_index("s")
    wid = s * NC + c
    rpt, rem = _tile_rows(n)
    nft = nchunks // tba
    last = nchunks - nft * tba
    base = wid * tba

    pltpu.sync_copy(zrows.at[pl.ds(0, rpt)], agg.at[pl.ds(s * rpt, rpt)])

    @pl.when(s == NS - 1)
    def _():
        pltpu.sync_copy(zrows.at[pl.ds(0, rem)], agg.at[pl.ds(NS * rpt, rem)])

    @pl.when(wid < nft)
    def _():
        pltpu.sync_copy(comb2.at[pl.ds(base, tba)], cidx.at[pl.ds(0, tba)])
    if last:
        @pl.when(wid == nft)
        def _():
            pltpu.sync_copy(comb2.at[pl.ds(base, last)], cidx.at[pl.ds(0, last)])

    plsc.subcore_barrier()

    def unpack(j, srow, drow):
        for i in range(CHUNK // 16):
            v = cidx[j, pl.ds(i * 16, 16)]
            srow[pl.ds(i * 16, 16)] = lax.bitwise_and(v, 0xFFFF)
            drow[pl.ds(i * 16, 16)] = jnp.minimum(
                lax.shift_right_logical(v, 16), n - 1)

    def gather(srow, buf):
        return pltpu.make_async_copy(y2.at[srow], buf, gsem)

    def scatter(buf, drow):
        return pltpu.make_async_copy(buf, agg.at[drow], ssem)

    cnt = jnp.maximum(0, jnp.minimum(tba, nchunks - base))
    half = cnt // 2

    @pl.when(half > 0)
    def _():
        unpack(0, srow_a, drow_a)
        gather(srow_a, rows_a).start()

        def body(k, carry):
            a = 2 * k
            b = a + 1
            unpack(b, srow_b, drow_b)
            gather(srow_a, rows_a).wait()
            sc_a = scatter(rows_a, drow_a)
            sc_a.start(add=True)
            g_b = gather(srow_b, rows_b)
            g_b.start()
            g_b.wait()
            sc_a.wait()
            sc_b = scatter(rows_b, drow_b)
            sc_b.start(add=True)

            @pl.when(k + 1 < half)
            def _():
                unpack(a + 2, srow_a, drow_a)
                gather(srow_a, rows_a).start()

            sc_b.wait()
            return carry

        lax.fori_loop(0, half, body, 0)

    plsc.subcore_barrier()

    pltpu.sync_copy(agg.at[pl.ds(s * rpt, rpt)],
                    out.at[c, pl.ds(s * rpt, rpt)])

    @pl.when(s == NS - 1)
    def _():
        pltpu.sync_copy(agg.at[pl.ds(NS * rpt, rem)],
                        out.at[c, pl.ds(NS * rpt, rem)])


def _sc_aggregate(comb2, y2, n_out):
    nchunks = comb2.shape[0]
    d = y2.shape[1]
    rpt, _ = _tile_rows(n_out)
    tba = -(-(-(-nchunks // NW)) // 8) * 8
    zrows = jnp.zeros((rpt, d), jnp.float32)
    mesh = plsc.VectorSubcoreMesh(core_axis_name="c", subcore_axis_name="s")
    return pl.kernel(
        functools.partial(_agg_body, nchunks, tba),
        out_type=jax.ShapeDtypeStruct((NC, n_out, d), jnp.float32),
        mesh=mesh,
        scratch_types=[
            pltpu.VMEM((tba, CHUNK), jnp.int32),
            pltpu.VMEM((CHUNK,), jnp.int32),
            pltpu.VMEM((CHUNK,), jnp.int32),
            pltpu.VMEM((CHUNK,), jnp.int32),
            pltpu.VMEM((CHUNK,), jnp.int32),
            pltpu.VMEM((CHUNK, d), jnp.float32),
            pltpu.VMEM((CHUNK, d), jnp.float32),
            pltpu.VMEM_SHARED((n_out, d), jnp.float32),
            pltpu.SemaphoreType.DMA,
            pltpu.SemaphoreType.DMA,
        ],
        compiler_params=pltpu.CompilerParams(needs_layout_passes=False),
    )(comb2, y2, zrows)


def _dinv(deg):
    return jnp.where(deg > 0, lax.rsqrt(jnp.maximum(deg, 1e-12)), 0.0)


def _matmul_body(x_ref, w_ref, y_ref):
    y_ref[...] = jnp.dot(x_ref[...], w_ref[...],
                         preferred_element_type=jnp.float32)


def _tc_matmul(x, w, n_pad, blk=2000):
    dout = w.shape[1]
    grid = -(-n_pad // blk)
    return pl.pallas_call(
        _matmul_body,
        grid=(grid,),
        in_specs=[
            pl.BlockSpec((blk, x.shape[1]), lambda i: (i, 0)),
            pl.BlockSpec((w.shape[0], dout), lambda i: (0, 0)),
        ],
        out_specs=pl.BlockSpec((blk, dout), lambda i: (i, 0)),
        out_shape=jax.ShapeDtypeStruct((n_pad, dout), jnp.float32),
    )(x, w)


def _scale_body(nv, blk, y_ref, hist_ref, y2_ref):
    i = pl.program_id(0)
    deg_src = hist_ref[0, 0] + hist_ref[1, 0]
    y2 = y_ref[...] * _dinv(deg_src)
    rows = i * blk + lax.broadcasted_iota(jnp.int32, y2.shape, 0)
    y2_ref[...] = jnp.where(rows < nv, y2, 0.0)


def _tc_scale(y, hist, n, blk=2000):
    n_pad, d = y.shape
    grid = -(-n_pad // blk)
    return pl.pallas_call(
        functools.partial(_scale_body, n, blk),
        grid=(grid,),
        in_specs=[
            pl.BlockSpec((blk, d), lambda i: (i, 0)),
            pl.BlockSpec((NC, 2, blk, 1), lambda i: (0, 0, i, 0)),
        ],
        out_specs=pl.BlockSpec((blk, d), lambda i: (i, 0)),
        out_shape=jax.ShapeDtypeStruct((n_pad, d), jnp.float32),
        input_output_aliases={0: 0},
    )(y, hist)


def _final_body(parts_ref, hist_ref, b_ref, out_ref):
    deg_dst = hist_ref[0, 1] + hist_ref[1, 1]
    agg = parts_ref[0] + parts_ref[1]
    out_ref[...] = agg * _dinv(deg_dst) + b_ref[...]


def _tc_final(parts, hist, b, blk=2000):
    n, d = parts.shape[1], parts.shape[2]
    grid = n // blk
    return pl.pallas_call(
        _final_body,
        grid=(grid,),
        in_specs=[
            pl.BlockSpec((NC, blk, d), lambda i: (0, i, 0)),
            pl.BlockSpec((NC, 2, blk, 1), lambda i: (0, 0, i, 0)),
            pl.BlockSpec((1, d), lambda i: (0, 0)),
        ],
        out_specs=pl.BlockSpec((blk, d), lambda i: (i, 0)),
        out_shape=jax.ShapeDtypeStruct((n, d), jnp.float32),
    )(parts, hist, b.reshape(1, d))


@jax.jit
def kernel(x, edge_index, W, b):
    n = x.shape[0]
    e = edge_index.shape[1]
    e_pad = -(-e // (8 * CHUNK)) * (8 * CHUNK)
    src_e, dst_e = edge_index[0], edge_index[1]
    comb = src_e | (dst_e << 16)
    if e_pad > e:
        comb = jnp.concatenate(
            [comb, jnp.full((e_pad - e,), n | (n << 16), dtype=comb.dtype)])
    comb2 = comb.reshape(e_pad // CHUNK, CHUNK)

    n_pad = n + 8 if e_pad > e else n
    hist = _sc_hist(comb2, n)
    y = _tc_matmul(x, W, n_pad)
    y2 = _tc_scale(y, hist, n)
    parts = _sc_aggregate(comb2, y2, n)
    return _tc_final(parts, hist, b)


# speedup vs baseline: 30.4302x; 1.0266x over previous
import functools

import jax
import jax.numpy as jnp
from jax import lax
from jax.experimental import pallas as pl
from jax.experimental.pallas import tpu as pltpu
from jax.experimental.pallas import tpu_sc as plsc

NC = 2
NS = 16
NW = NC * NS
CHUNK = 128
TB = 80


def _tile_rows(n):
    rpt = (n // NS) // 8 * 8
    rem = n - rpt * NS
    return rpt, rem


def _hist_body(nchunks, comb2, zeros_h, row_ids, out,
               cidx, hs, hd, ridx, sh_s, sh_d):
    nrows = sh_s.shape[0]
    c = lax.axis_index("c")
    s = lax.axis_index("s")
    wid = s * NC + c
    nw8 = nrows // 8
    nft = nchunks // TB
    last = nchunks - nft * TB
    base = wid * TB

    pltpu.sync_copy(zeros_h, hs)
    pltpu.sync_copy(zeros_h, hd)

    @pl.when(s < nw8)
    def _():
        pltpu.sync_copy(zeros_h.at[pl.ds(0, 8)], sh_s.at[pl.ds(s * 8, 8)])
        pltpu.sync_copy(zeros_h.at[pl.ds(0, 8)], sh_d.at[pl.ds(s * 8, 8)])

    pltpu.sync_copy(row_ids, ridx)

    @pl.when(wid < nft)
    def _():
        pltpu.sync_copy(comb2.at[pl.ds(base, TB)], cidx.at[pl.ds(0, TB)])
    if last:
        @pl.when(wid == nft)
        def _():
            pltpu.sync_copy(comb2.at[pl.ds(base, last)], cidx.at[pl.ds(0, last)])

    plsc.subcore_barrier()

    ones16 = jnp.ones((16,), jnp.float32)

    def scat(hist, iv):
        plsc.addupdate_scatter(
            hist, [lax.shift_right_logical(iv, 7), lax.bitwise_and(iv, 127)],
            ones16)

    cnt = jnp.maximum(0, jnp.minimum(TB, nchunks - base))

    def body(a, carry):
        for j in range(CHUNK // 16):
            v = cidx[a, pl.ds(j * 16, 16)]
            scat(hs, lax.bitwise_and(v, 0xFFFF))
            scat(hd, lax.shift_right_logical(v, 16))
        return carry

    lax.fori_loop(0, cnt, body, 0)

    pltpu.sync_copy(hs, sh_s.at[ridx], add=True)
    pltpu.sync_copy(hd, sh_d.at[ridx], add=True)
    plsc.subcore_barrier()

    @pl.when(s < nw8)
    def _():
        pltpu.sync_copy(sh_s.at[pl.ds(s * 8, 8)], out.at[c, 0, pl.ds(s * 8, 8)])
        pltpu.sync_copy(sh_d.at[pl.ds(s * 8, 8)], out.at[c, 1, pl.ds(s * 8, 8)])


def _sc_hist(comb2, n):
    nchunks = comb2.shape[0]
    nrows = -(-n // (128 * NS)) * NS
    zeros_h = jnp.zeros((nrows, 128), jnp.float32)
    row_ids = jnp.arange(nrows, dtype=jnp.int32)
    mesh = plsc.VectorSubcoreMesh(core_axis_name="c", subcore_axis_name="s")
    hist = pl.kernel(
        functools.partial(_hist_body, nchunks),
        out_type=jax.ShapeDtypeStruct((NC, 2, nrows, 128), jnp.float32),
        mesh=mesh,
        scratch_types=[
            pltpu.VMEM((TB, CHUNK), jnp.int32),
            pltpu.VMEM((nrows, 128), jnp.float32),
            pltpu.VMEM((nrows, 128), jnp.float32),
            pltpu.VMEM((nrows,), jnp.int32),
            pltpu.VMEM_SHARED((nrows, 128), jnp.float32),
            pltpu.VMEM_SHARED((nrows, 128), jnp.float32),
        ],
        compiler_params=pltpu.CompilerParams(needs_layout_passes=False),
    )(comb2, zeros_h, row_ids)
    return hist.reshape(NC, 2, 1, nrows * 128)


def _agg_body(nchunks, tba, comb2, y2, zrows, out,
              cidx, srow_a, drow_a, srow_b, drow_b, rows_a, rows_b,
              agg, gsem, ssem):
    n, d = agg.shape
    c = lax.axis_index("c")
    s = lax.axis_index("s")
    wid = s * NC + c
    rpt, rem = _tile_rows(n)
    nft = nchunks // tba
    last = nchunks - nft * tba
    base = wid * tba

    pltpu.sync_copy(zrows.at[pl.ds(0, rpt)], agg.at[pl.ds(s * rpt, rpt)])

    @pl.when(s == NS - 1)
    def _():
        pltpu.sync_copy(zrows.at[pl.ds(0, rem)], agg.at[pl.ds(NS * rpt, rem)])

    @pl.when(wid < nft)
    def _():
        pltpu.sync_copy(comb2.at[pl.ds(base, tba)], cidx.at[pl.ds(0, tba)])
    if last:
        @pl.when(wid == nft)
        def _():
            pltpu.sync_copy(comb2.at[pl.ds(base, last)], cidx.at[pl.ds(0, last)])

    plsc.subcore_barrier()

    def unpack(j, srow, drow):
        for i in range(CHUNK // 16):
            v = cidx[j, pl.ds(i * 16, 16)]
            srow[pl.ds(i * 16, 16)] = lax.bitwise_and(v, 0xFFFF)
            drow[pl.ds(i * 16, 16)] = jnp.minimum(
                lax.shift_right_logical(v, 16), n - 1)

    def gather(srow, buf):
        return pltpu.make_async_copy(y2.at[srow], buf, gsem)

    def scatter(buf, drow):
        return pltpu.make_async_copy(buf, agg.at[drow], ssem)

    cnt = jnp.maximum(0, jnp.minimum(tba, nchunks - base))
    half = cnt // 2

    @pl.when(half > 0)
    def _():
        unpack(0, srow_a, drow_a)
        gather(srow_a, rows_a).start()

        def body(k, carry):
            a = 2 * k
            b = a + 1
            unpack(b, srow_b, drow_b)
            gather(srow_a, rows_a).wait()
            sc_a = scatter(rows_a, drow_a)
            sc_a.start(add=True)
            g_b = gather(srow_b, rows_b)
            g_b.start()
            g_b.wait()
            sc_a.wait()
            sc_b = scatter(rows_b, drow_b)
            sc_b.start(add=True)

            @pl.when(k + 1 < half)
            def _():
                unpack(a + 2, srow_a, drow_a)
                gather(srow_a, rows_a).start()

            sc_b.wait()
            return carry

        lax.fori_loop(0, half, body, 0)

    plsc.subcore_barrier()

    pltpu.sync_copy(agg.at[pl.ds(s * rpt, rpt)],
                    out.at[c, pl.ds(s * rpt, rpt)])

    @pl.when(s == NS - 1)
    def _():
        pltpu.sync_copy(agg.at[pl.ds(NS * rpt, rem)],
                        out.at[c, pl.ds(NS * rpt, rem)])


def _sc_aggregate(comb2, y2, n_out):
    nchunks = comb2.shape[0]
    d = y2.shape[1]
    rpt, _ = _tile_rows(n_out)
    tba = -(-(-(-nchunks // NW)) // 8) * 8
    zrows = jnp.zeros((rpt, d), jnp.float32)
    mesh = plsc.VectorSubcoreMesh(core_axis_name="c", subcore_axis_name="s")
    return pl.kernel(
        functools.partial(_agg_body, nchunks, tba),
        out_type=jax.ShapeDtypeStruct((NC, n_out, d), jnp.float32),
        mesh=mesh,
        scratch_types=[
            pltpu.VMEM((tba, CHUNK), jnp.int32),
            pltpu.VMEM((CHUNK,), jnp.int32),
            pltpu.VMEM((CHUNK,), jnp.int32),
            pltpu.VMEM((CHUNK,), jnp.int32),
            pltpu.VMEM((CHUNK,), jnp.int32),
            pltpu.VMEM((CHUNK, d), jnp.float32),
            pltpu.VMEM((CHUNK, d), jnp.float32),
            pltpu.VMEM_SHARED((n_out, d), jnp.float32),
            pltpu.SemaphoreType.DMA,
            pltpu.SemaphoreType.DMA,
        ],
        compiler_params=pltpu.CompilerParams(needs_layout_passes=False),
    )(comb2, y2, zrows)


def _dinv(deg):
    return jnp.where(deg > 0, lax.rsqrt(jnp.maximum(deg, 1e-12)), 0.0)


def _matmul_body(x_ref, w_ref, y_ref):
    y_ref[...] = jnp.dot(x_ref[...], w_ref[...],
                         preferred_element_type=jnp.float32)


def _tc_matmul(x, w, n_pad, blk=2048):
    dout = w.shape[1]
    grid = -(-n_pad // blk)
    return pl.pallas_call(
        _matmul_body,
        grid=(grid,),
        in_specs=[
            pl.BlockSpec((blk, x.shape[1]), lambda i: (i, 0)),
            pl.BlockSpec((w.shape[0], dout), lambda i: (0, 0)),
        ],
        out_specs=pl.BlockSpec((blk, dout), lambda i: (i, 0)),
        out_shape=jax.ShapeDtypeStruct((n_pad, dout), jnp.float32),
    )(x, w)


def _col(ident_ref, row):
    return lax.dot_general(ident_ref[...], row, (((1,), (1,)), ((), ())),
                           preferred_element_type=jnp.float32)


def _scale_body(nv, blk, y_ref, hist_ref, ident_ref, y2_ref):
    i = pl.program_id(0)
    deg_src = _col(ident_ref, hist_ref[0, 0] + hist_ref[1, 0])
    y2 = y_ref[...] * _dinv(deg_src)
    rows = i * blk + lax.broadcasted_iota(jnp.int32, y2.shape, 0)
    y2_ref[...] = jnp.where(rows < nv, y2, 0.0)


def _tc_scale(y, hist, ident, n):
    n_pad, d = y.shape
    blk = ident.shape[0]
    grid = -(-n_pad // blk)
    return pl.pallas_call(
        functools.partial(_scale_body, n, blk),
        grid=(grid,),
        in_specs=[
            pl.BlockSpec((blk, d), lambda i: (i, 0)),
            pl.BlockSpec((NC, 2, 1, blk), lambda i: (0, 0, 0, i)),
            pl.BlockSpec((blk, blk), lambda i: (0, 0)),
        ],
        out_specs=pl.BlockSpec((blk, d), lambda i: (i, 0)),
        out_shape=jax.ShapeDtypeStruct((n_pad, d), jnp.float32),
        input_output_aliases={0: 0},
    )(y, hist, ident)


def _final_body(parts_ref, hist_ref, ident_ref, b_ref, out_ref):
    deg_dst = _col(ident_ref, hist_ref[0, 1] + hist_ref[1, 1])
    agg = parts_ref[0] + parts_ref[1]
    out_ref[...] = agg * _dinv(deg_dst) + b_ref[...]


def _tc_final(parts, hist, ident, b):
    n, d = parts.shape[1], parts.shape[2]
    blk = ident.shape[0]
    grid = -(-n // blk)
    return pl.pallas_call(
        _final_body,
        grid=(grid,),
        in_specs=[
            pl.BlockSpec((NC, blk, d), lambda i: (0, i, 0)),
            pl.BlockSpec((NC, 2, 1, blk), lambda i: (0, 0, 0, i)),
            pl.BlockSpec((blk, blk), lambda i: (0, 0)),
            pl.BlockSpec((1, d), lambda i: (0, 0)),
        ],
        out_specs=pl.BlockSpec((blk, d), lambda i: (i, 0)),
        out_shape=jax.ShapeDtypeStruct((n, d), jnp.float32),
    )(parts, hist, ident, b.reshape(1, d))


@jax.jit
def kernel(x, edge_index, W, b):
    n = x.shape[0]
    e = edge_index.shape[1]
    e_pad = -(-e // (8 * CHUNK)) * (8 * CHUNK)
    src_e, dst_e = edge_index[0], edge_index[1]
    comb = src_e | (dst_e << 16)
    if e_pad > e:
        comb = jnp.concatenate(
            [comb, jnp.full((e_pad - e,), n | (n << 16), dtype=comb.dtype)])
    comb2 = comb.reshape(e_pad // CHUNK, CHUNK)

    n_pad = -(-n // (128 * NS)) * NS * 128
    ident = jnp.eye(512, dtype=jnp.float32)
    hist = _sc_hist(comb2, n)
    y = _tc_matmul(x, W, n_pad)
    y2 = _tc_scale(y, hist, ident, n)
    parts = _sc_aggregate(comb2, y2, n)
    return _tc_final(parts, hist, ident, b)
